# Initial kernel scaffold; baseline (speedup 1.0000x reference)
import jax
import jax.numpy as jnp
from jax.experimental import pallas as pl


def kernel(output, target, extra_len):
    raise NotImplementedError("write your pallas kernel here")


# speedup vs baseline: 2.4737x; 2.4737x over previous
import math

import jax
import jax.numpy as jnp
from jax.experimental import pallas as pl
from jax.experimental.pallas import tpu as pltpu

_LS = 0.1
_CONF = 1.0 - _LS
_BLK = 2048


def _body(nblocks, blk, n, t_ref, x_ref, out_ref, acc_ref, tacc_ref, zacc_ref):
    j = pl.program_id(0)

    @pl.when(j == 0)
    def _init():
        acc_ref[...] = jnp.zeros_like(acc_ref)
        tacc_ref[...] = jnp.zeros_like(tacc_ref)

    x = x_ref[...]
    cols = j * blk + jax.lax.broadcasted_iota(jnp.int32, x.shape, 1)
    t = t_ref[...]
    zero = jnp.zeros_like(x)
    acc_ref[...] += jnp.sum(jnp.where(cols < n, x, zero), axis=1, keepdims=True)
    tacc_ref[...] += jnp.sum(jnp.where(cols == t, x, zero), axis=1, keepdims=True)

    @pl.when(j == 0)
    def _zcol():
        zacc_ref[...] = x[:, 0:1]

    @pl.when(j == nblocks - 1)
    def _finish():
        s = _LS / (n - 2)
        k_const = _LS * math.log(s) + _CONF * math.log(_CONF)
        contrib = (k_const
                   - s * acc_ref[...]
                   + s * zacc_ref[...]
                   + (s - _CONF) * tacc_ref[...])
        nonpad = t_ref[...] != 0
        total = jnp.sum(jnp.where(nonpad, contrib, jnp.zeros_like(contrib)))
        out_ref[...] = total.reshape(1, 1)


def kernel(output, target, extra_len):
    del extra_len
    b, n = output.shape
    nblocks = pl.cdiv(n, _BLK)
    t2 = target.astype(jnp.int32).reshape(b, 1)

    import functools
    body = functools.partial(_body, nblocks, _BLK, n)
    res = pl.pallas_call(
        body,
        grid=(nblocks,),
        in_specs=[
            pl.BlockSpec((b, 1), lambda j: (0, 0)),
            pl.BlockSpec((b, _BLK), lambda j: (0, j)),
        ],
        out_specs=pl.BlockSpec((1, 1), lambda j: (0, 0)),
        out_shape=jax.ShapeDtypeStruct((1, 1), jnp.float32),
        scratch_shapes=[
            pltpu.VMEM((b, 1), jnp.float32),
            pltpu.VMEM((b, 1), jnp.float32),
            pltpu.VMEM((b, 1), jnp.float32),
        ],
    )(t2, output)
    return res[0, 0]


# speedup vs baseline: 2.4765x; 1.0011x over previous
import math

import jax
import jax.numpy as jnp
from jax.experimental import pallas as pl
from jax.experimental.pallas import tpu as pltpu

_LS = 0.1
_CONF = 1.0 - _LS
_BLK = 4096


def _body(nblocks, blk, n, t_ref, x_ref, out_ref, acc_ref, tacc_ref, zacc_ref):
    j = pl.program_id(0)

    @pl.when(j == 0)
    def _init():
        acc_ref[...] = jnp.zeros_like(acc_ref)
        tacc_ref[...] = jnp.zeros_like(tacc_ref)

    x = x_ref[...]
    cols = j * blk + jax.lax.broadcasted_iota(jnp.int32, x.shape, 1)
    t = t_ref[...]
    zero = jnp.zeros_like(x)

    @pl.when(j < nblocks - 1)
    def _full():
        acc_ref[...] += jnp.sum(x, axis=1, keepdims=True)

    @pl.when(j == nblocks - 1)
    def _masked():
        acc_ref[...] += jnp.sum(jnp.where(cols < n, x, zero), axis=1,
                                keepdims=True)

    tacc_ref[...] += jnp.sum(jnp.where(cols == t, x, zero), axis=1, keepdims=True)

    @pl.when(j == 0)
    def _zcol():
        zacc_ref[...] = x[:, 0:1]

    @pl.when(j == nblocks - 1)
    def _finish():
        s = _LS / (n - 2)
        k_const = _LS * math.log(s) + _CONF * math.log(_CONF)
        contrib = (k_const
                   - s * acc_ref[...]
                   + s * zacc_ref[...]
                   + (s - _CONF) * tacc_ref[...])
        nonpad = t_ref[...] != 0
        total = jnp.sum(jnp.where(nonpad, contrib, jnp.zeros_like(contrib)))
        out_ref[...] = total.reshape(1, 1)


def kernel(output, target, extra_len):
    del extra_len
    b, n = output.shape
    nblocks = pl.cdiv(n, _BLK)
    t2 = target.astype(jnp.int32).reshape(b, 1)

    import functools
    body = functools.partial(_body, nblocks, _BLK, n)
    res = pl.pallas_call(
        body,
        grid=(nblocks,),
        in_specs=[
            pl.BlockSpec((b, 1), lambda j: (0, 0)),
            pl.BlockSpec((b, _BLK), lambda j: (0, j)),
        ],
        out_specs=pl.BlockSpec((1, 1), lambda j: (0, 0)),
        out_shape=jax.ShapeDtypeStruct((1, 1), jnp.float32),
        scratch_shapes=[
            pltpu.VMEM((b, 1), jnp.float32),
            pltpu.VMEM((b, 1), jnp.float32),
            pltpu.VMEM((b, 1), jnp.float32),
        ],
    )(t2, output)
    return res[0, 0]
